# baseline (speedup 1.0000x reference)
import numpy as np
import jax
import jax.numpy as jnp
from jax import lax
from jax.experimental import pallas as pl
from jax.experimental.pallas import tpu as pltpu
from jax.experimental.pallas import tpu_sc as plsc

IMG_SIZE = 640
STRIDES = (8, 16, 32, 64, 128)
LIMITS = ((-1.0, 64.0), (64.0, 128.0), (128.0, 256.0), (256.0, 512.0),
          (512.0, 999999.0))
RATIO = 1.5

B = 16
NGT = 64
LVL_SIZES = tuple((IMG_SIZE // s) * (IMG_SIZE // s) for s in STRIDES)
LVL_PAD = tuple(-(-a // 16) * 16 for a in LVL_SIZES)
NWORK = 32
TOT_VECS = -(-sum(LVL_PAD) // (16 * NWORK)) * NWORK
PAD_A = TOT_VECS * 16
KVECS = TOT_VECS // NWORK
STRIPE = KVECS * 16
VEC_BOUNDS = tuple(np.cumsum([p // 16 for p in LVL_PAD]))
IMAX = 0x7FFFFFFF


def _build_params():
    p = np.zeros((5, PAD_A), np.float32)
    off = 0
    for s, (lo, hi), a, ap in zip(STRIDES, LIMITS, LVL_SIZES, LVL_PAD):
        w = IMG_SIZE // s
        idx = np.arange(a)
        p[0, off:off + a] = (idx % w) * s + s // 2
        p[1, off:off + a] = (idx // w) * s + s // 2
        p[2, off:off + a] = lo
        p[3, off:off + a] = hi
        p[4, off:off + a] = 2.0 * s * RATIO
        off += ap
    return p.reshape(-1)


_PARAMS = _build_params()


def _body(boxes_hbm, labels_hbm, params_hbm,
          cls_hbm, cnt_hbm, reg_hbm,
          params_vm, box_vm, lab_vm,
          x0_vm, y0_vm, x1_vm, y1_vm, key_vm,
          cls_o, cnt_o, rl_o, rt_o, rr_o, rb_o, sem):
    w = lax.axis_index("s") * 2 + lax.axis_index("c")
    abase = w * STRIPE

    in_h = [pltpu.async_copy(params_hbm.at[pl.ds(i * PAD_A + abase, STRIPE)],
                             params_vm.at[pl.ds(i * STRIPE, STRIPE)], sem)
            for i in range(5)]
    in_h.append(pltpu.async_copy(boxes_hbm, box_vm, sem))
    in_h.append(pltpu.async_copy(labels_hbm, lab_vm, sem))
    for h in in_h:
        h.wait()

    los = [jnp.float32(lo) for lo, _ in LIMITS] + [jnp.float32(0.0)]
    his2 = [jnp.float32(2.0 * hi) for _, hi in LIMITS] + [jnp.float32(0.0)]

    def bbody(b, _):
        gtb = b * NGT
        ms, rms = [], []
        for i in range(NGT // 16):
            sl = pl.ds(gtb + i * 16, 16)
            x0 = box_vm[pl.ds(b * 4 * NGT + 0 * NGT + i * 16, 16)]
            y0 = box_vm[pl.ds(b * 4 * NGT + 1 * NGT + i * 16, 16)]
            wv = box_vm[pl.ds(b * 4 * NGT + 2 * NGT + i * 16, 16)]
            hv = box_vm[pl.ds(b * 4 * NGT + 3 * NGT + i * 16, 16)]
            x0_vm[sl] = x0
            y0_vm[sl] = y0
            x1_vm[sl] = x0 + wv
            y1_vm[sl] = y0 + hv
            jidx = lax.iota(jnp.int32, 16) + (i * 16)
            key_vm[sl] = (wv * hv).astype(jnp.int32) * 64 + jidx
            m = jnp.maximum(wv, hv)
            ms.append(m)
            rms.append(lax.rev(m, (0,)))

        firsts, lasts = [], []
        for lvl in range(5):
            lo, hi2 = los[lvl], his2[lvl]
            fv, lv = [], []
            for c in range(NGT // 16):
                cand = (ms[c] > lo) & (ms[c] <= hi2)
                fc = plsc.all_reduce_ffs(cand)
                fv.append(jnp.where(fc >= 16, NGT, fc + (c * 16)))
                candr = (rms[c] > lo) & (rms[c] <= hi2)
                fr = plsc.all_reduce_ffs(candr)
                lv.append(jnp.where(fr >= 16, 0, (c * 16 + 16) - fr))
            firstv = jnp.minimum(jnp.minimum(fv[0], fv[1]),
                                 jnp.minimum(fv[2], fv[3]))
            lastv = jnp.maximum(jnp.maximum(lv[0], lv[1]),
                                jnp.maximum(lv[2], lv[3]))
            firsts.append(firstv[0])
            lasts.append(lastv[0])
        firsts.append(jnp.int32(0))
        lasts.append(jnp.int32(0))

        def kbody(k, _):
            g = w * KVECS + k
            lvl = ((g >= VEC_BOUNDS[0]).astype(jnp.int32)
                   + (g >= VEC_BOUNDS[1]) + (g >= VEC_BOUNDS[2])
                   + (g >= VEC_BOUNDS[3]) + (g >= VEC_BOUNDS[4]))
            first = firsts[5]
            last = lasts[5]
            for i in range(4, -1, -1):
                first = jnp.where(lvl == i, firsts[i], first)
                last = jnp.where(lvl == i, lasts[i], last)

            osl = pl.ds(b * STRIPE + k * 16, 16)

            @pl.when(last <= first)
            def _empty():
                cls_o[osl] = jnp.full((16,), -1, jnp.int32)
                cnt_o[osl] = jnp.full((16,), -1.0, jnp.float32)
                rl_o[osl] = jnp.full((16,), -1.0, jnp.float32)
                rt_o[osl] = jnp.full((16,), -1.0, jnp.float32)
                rr_o[osl] = jnp.full((16,), -1.0, jnp.float32)
                rb_o[osl] = jnp.full((16,), -1.0, jnp.float32)

            @pl.when(last > first)
            def _work():
                xv = params_vm[pl.ds(0 * STRIPE + k * 16, 16)]
                yv = params_vm[pl.ds(1 * STRIPE + k * 16, 16)]
                lov = params_vm[pl.ds(2 * STRIPE + k * 16, 16)]
                hiv = params_vm[pl.ds(3 * STRIPE + k * 16, 16)]
                radv = params_vm[pl.ds(4 * STRIPE + k * 16, 16)]

                def one(jv, valid):
                    x0s = plsc.load_gather(x0_vm, [jv])
                    y0s = plsc.load_gather(y0_vm, [jv])
                    x1s = plsc.load_gather(x1_vm, [jv])
                    y1s = plsc.load_gather(y1_vm, [jv])
                    keys = plsc.load_gather(key_vm, [jv])
                    l = xv - x0s
                    t = yv - y0s
                    r = x1s - xv
                    bb = y1s - yv
                    omin = jnp.minimum(jnp.minimum(l, r), jnp.minimum(t, bb))
                    omax = jnp.maximum(jnp.maximum(l, r), jnp.maximum(t, bb))
                    dmax = jnp.maximum(jnp.abs(l - r), jnp.abs(t - bb))
                    pos = ((omin > 0.0) & (omax > lov) & (omax <= hiv)
                           & (dmax < radv))
                    if valid is not None:
                        pos = pos & valid
                    return jnp.where(pos, keys, IMAX)

                base_j = gtb + first
                lastv = jnp.full((16,), gtb + last, jnp.int32)

                def jbody(i, best):
                    jv0 = jnp.full((16,), base_j + i * 4, jnp.int32)
                    jv1 = jv0 + 1
                    jv2 = jv0 + 2
                    jv3 = jv0 + 3
                    am0 = one(jv0, None)
                    am1 = one(jv1, jv1 < lastv)
                    am2 = one(jv2, jv2 < lastv)
                    am3 = one(jv3, jv3 < lastv)
                    am = jnp.minimum(jnp.minimum(am0, am1),
                                     jnp.minimum(am2, am3))
                    return jnp.minimum(best, am)

                best = lax.fori_loop(0, (last - first + 3) // 4, jbody,
                                     jnp.full((16,), 0x7FFFFFFF, jnp.int32))
                anyv = best < IMAX
                bj = (best & 63) + gtb

                bx0 = plsc.load_gather(x0_vm, [bj])
                by0 = plsc.load_gather(y0_vm, [bj])
                bx1 = plsc.load_gather(x1_vm, [bj])
                by1 = plsc.load_gather(y1_vm, [bj])
                labg = plsc.load_gather(lab_vm, [bj])

                lg = xv - bx0
                tg = yv - by0
                rg = bx1 - xv
                bg = by1 - yv
                lrmin = jnp.minimum(lg, rg)
                lrmax = jnp.maximum(lg, rg)
                tbmin = jnp.minimum(tg, bg)
                tbmax = jnp.maximum(tg, bg)
                ratio = (lrmin * tbmin) / (lrmax * tbmax + 1e-10)
                rs = jnp.where(anyv, ratio, 1.0)
                rs = jnp.maximum(rs, 1e-30)
                ii = lax.bitcast_convert_type(rs, jnp.int32)
                r0 = lax.bitcast_convert_type(0x5F3759DF - (ii >> 1),
                                              jnp.float32)
                for _ in range(3):
                    r0 = r0 * (1.5 - 0.5 * rs * r0 * r0)
                sq = rs * r0

                cls_o[osl] = jnp.where(anyv, labg, -1)
                cnt_o[osl] = jnp.where(anyv, sq, -1.0)
                rl_o[osl] = jnp.where(anyv, lg, -1.0)
                rt_o[osl] = jnp.where(anyv, tg, -1.0)
                rr_o[osl] = jnp.where(anyv, rg, -1.0)
                rb_o[osl] = jnp.where(anyv, bg, -1.0)

            return 0

        lax.fori_loop(0, KVECS, kbody, 0)
        return 0

    lax.fori_loop(0, B, bbody, 0)

    out_h = []
    for b in range(B):
        src = pl.ds(b * STRIPE, STRIPE)
        dst = pl.ds(b * PAD_A + abase, STRIPE)
        out_h.append(pltpu.async_copy(cls_o.at[src], cls_hbm.at[dst], sem))
        out_h.append(pltpu.async_copy(cnt_o.at[src], cnt_hbm.at[dst], sem))
        for c, r_o in enumerate((rl_o, rt_o, rr_o, rb_o)):
            out_h.append(pltpu.async_copy(
                r_o.at[src],
                reg_hbm.at[pl.ds(c * B * PAD_A + b * PAD_A + abase, STRIPE)],
                sem))
    for h in out_h:
        h.wait()


@jax.jit
def _assign(boxes_t, labels):
    f = pl.kernel(
        _body,
        out_type=[
            jax.ShapeDtypeStruct((B * PAD_A,), jnp.int32),
            jax.ShapeDtypeStruct((B * PAD_A,), jnp.float32),
            jax.ShapeDtypeStruct((4 * B * PAD_A,), jnp.float32),
        ],
        mesh=plsc.VectorSubcoreMesh(core_axis_name="c", subcore_axis_name="s",
                                    num_cores=2, num_subcores=16),
        compiler_params=pltpu.CompilerParams(needs_layout_passes=False),
        scratch_types=(
            [pltpu.VMEM((5 * STRIPE,), jnp.float32),
             pltpu.VMEM((B * 4 * NGT,), jnp.float32),
             pltpu.VMEM((B * NGT,), jnp.int32)]
            + [pltpu.VMEM((B * NGT + 16,), jnp.float32)] * 4
            + [pltpu.VMEM((B * NGT + 16,), jnp.int32)]
            + [pltpu.VMEM((B * STRIPE,), jnp.int32),
               pltpu.VMEM((B * STRIPE,), jnp.float32)]
            + [pltpu.VMEM((B * STRIPE,), jnp.float32)] * 4
            + [pltpu.SemaphoreType.DMA]
        ),
    )
    return f(boxes_t.reshape(-1), labels.reshape(-1), _PARAMS)


def kernel(gt_boxes, gt_labels):
    boxes_t = jnp.transpose(gt_boxes, (0, 2, 1))
    labels = gt_labels.astype(jnp.int32)
    cls_f, cnt_f, reg_f = _assign(boxes_t, labels)
    cls_f = cls_f.reshape(B, PAD_A)
    cnt_f = cnt_f.reshape(B, PAD_A)
    reg_f = reg_f.reshape(4, B, PAD_A)

    offs = np.cumsum([0] + list(LVL_PAD))[:-1]
    cls = jnp.concatenate(
        [cls_f[:, o:o + a] for o, a in zip(offs, LVL_SIZES)], axis=1)
    cnt = jnp.concatenate(
        [cnt_f[:, o:o + a] for o, a in zip(offs, LVL_SIZES)], axis=1)
    reg = jnp.concatenate(
        [reg_f[:, :, o:o + a] for o, a in zip(offs, LVL_SIZES)], axis=2)
    cls = cls.reshape(-1, 1)
    cnt = cnt.reshape(-1, 1)
    reg = reg.reshape(4, -1).T
    return cls, cnt, reg


# speedup vs baseline: 1.0061x; 1.0061x over previous
import numpy as np
import jax
import jax.numpy as jnp
from jax import lax
from jax.experimental import pallas as pl
from jax.experimental.pallas import tpu as pltpu
from jax.experimental.pallas import tpu_sc as plsc

IMG_SIZE = 640
STRIDES = (8, 16, 32, 64, 128)
LIMITS = ((-1.0, 64.0), (64.0, 128.0), (128.0, 256.0), (256.0, 512.0),
          (512.0, 999999.0))
RATIO = 1.5

B = 16
NGT = 64
LVL_SIZES = tuple((IMG_SIZE // s) * (IMG_SIZE // s) for s in STRIDES)
LVL_PAD = tuple(-(-a // 16) * 16 for a in LVL_SIZES)
NWORK = 32
TOT_VECS = -(-sum(LVL_PAD) // (16 * NWORK)) * NWORK
PAD_A = TOT_VECS * 16
KVECS = TOT_VECS // NWORK
STRIPE = KVECS * 16
VEC_BOUNDS = tuple(np.cumsum([p // 16 for p in LVL_PAD]))
IMAX = 0x7FFFFFFF


def _build_params():
    p = np.zeros((5, PAD_A), np.float32)
    off = 0
    for s, (lo, hi), a, ap in zip(STRIDES, LIMITS, LVL_SIZES, LVL_PAD):
        w = IMG_SIZE // s
        idx = np.arange(a)
        p[0, off:off + a] = (idx % w) * s + s // 2
        p[1, off:off + a] = (idx // w) * s + s // 2
        p[2, off:off + a] = lo
        p[3, off:off + a] = hi
        p[4, off:off + a] = 2.0 * s * RATIO
        off += ap
    return p.reshape(-1)


_PARAMS = _build_params()


def _body(boxes_hbm, labels_hbm, params_hbm,
          cls_hbm, cnt_hbm, reg_hbm,
          params_vm, box_vm, lab_vm,
          x0_vm, y0_vm, x1_vm, y1_vm, key_vm,
          cls_o, cnt_o, rl_o, rt_o, rr_o, rb_o, sem):
    w = lax.axis_index("s") * 2 + lax.axis_index("c")
    abase = w * STRIPE

    in_h = [pltpu.async_copy(params_hbm.at[pl.ds(i * PAD_A + abase, STRIPE)],
                             params_vm.at[pl.ds(i * STRIPE, STRIPE)], sem)
            for i in range(5)]
    in_h.append(pltpu.async_copy(boxes_hbm, box_vm, sem))
    in_h.append(pltpu.async_copy(labels_hbm, lab_vm, sem))
    for h in in_h:
        h.wait()

    los = [jnp.float32(lo) for lo, _ in LIMITS] + [jnp.float32(0.0)]
    his2 = [jnp.float32(2.0 * hi) for _, hi in LIMITS] + [jnp.float32(0.0)]

    def bbody(b, _):
        gtb = b * NGT
        ms, rms = [], []
        for i in range(NGT // 16):
            sl = pl.ds(gtb + i * 16, 16)
            x0 = box_vm[pl.ds(b * 4 * NGT + 0 * NGT + i * 16, 16)]
            y0 = box_vm[pl.ds(b * 4 * NGT + 1 * NGT + i * 16, 16)]
            wv = box_vm[pl.ds(b * 4 * NGT + 2 * NGT + i * 16, 16)]
            hv = box_vm[pl.ds(b * 4 * NGT + 3 * NGT + i * 16, 16)]
            x0_vm[sl] = x0
            y0_vm[sl] = y0
            x1_vm[sl] = x0 + wv
            y1_vm[sl] = y0 + hv
            jidx = lax.iota(jnp.int32, 16) + (i * 16)
            key_vm[sl] = (wv * hv).astype(jnp.int32) * 64 + jidx
            m = jnp.maximum(wv, hv)
            ms.append(m)
            rms.append(lax.rev(m, (0,)))

        firsts, lasts = [], []
        for lvl in range(5):
            lo, hi2 = los[lvl], his2[lvl]
            fv, lv = [], []
            for c in range(NGT // 16):
                cand = (ms[c] > lo) & (ms[c] <= hi2)
                fc = plsc.all_reduce_ffs(cand)
                fv.append(jnp.where(fc >= 16, NGT, fc + (c * 16)))
                candr = (rms[c] > lo) & (rms[c] <= hi2)
                fr = plsc.all_reduce_ffs(candr)
                lv.append(jnp.where(fr >= 16, 0, (c * 16 + 16) - fr))
            firstv = jnp.minimum(jnp.minimum(fv[0], fv[1]),
                                 jnp.minimum(fv[2], fv[3]))
            lastv = jnp.maximum(jnp.maximum(lv[0], lv[1]),
                                jnp.maximum(lv[2], lv[3]))
            firsts.append(firstv[0])
            lasts.append(lastv[0])
        firsts.append(jnp.int32(0))
        lasts.append(jnp.int32(0))

        def kbody(k, _):
            g = w * KVECS + k
            lvl = ((g >= VEC_BOUNDS[0]).astype(jnp.int32)
                   + (g >= VEC_BOUNDS[1]) + (g >= VEC_BOUNDS[2])
                   + (g >= VEC_BOUNDS[3]) + (g >= VEC_BOUNDS[4]))
            first = firsts[5]
            last = lasts[5]
            for i in range(4, -1, -1):
                first = jnp.where(lvl == i, firsts[i], first)
                last = jnp.where(lvl == i, lasts[i], last)

            osl = pl.ds(b * STRIPE + k * 16, 16)

            @pl.when(last <= first)
            def _empty():
                cls_o[osl] = jnp.full((16,), -1, jnp.int32)
                cnt_o[osl] = jnp.full((16,), -1.0, jnp.float32)
                rl_o[osl] = jnp.full((16,), -1.0, jnp.float32)
                rt_o[osl] = jnp.full((16,), -1.0, jnp.float32)
                rr_o[osl] = jnp.full((16,), -1.0, jnp.float32)
                rb_o[osl] = jnp.full((16,), -1.0, jnp.float32)

            @pl.when(last > first)
            def _work():
                xv = params_vm[pl.ds(0 * STRIPE + k * 16, 16)]
                yv = params_vm[pl.ds(1 * STRIPE + k * 16, 16)]
                lov = params_vm[pl.ds(2 * STRIPE + k * 16, 16)]
                hiv = params_vm[pl.ds(3 * STRIPE + k * 16, 16)]
                radv = params_vm[pl.ds(4 * STRIPE + k * 16, 16)]

                def one(jv, valid):
                    x0s = plsc.load_gather(x0_vm, [jv])
                    y0s = plsc.load_gather(y0_vm, [jv])
                    x1s = plsc.load_gather(x1_vm, [jv])
                    y1s = plsc.load_gather(y1_vm, [jv])
                    keys = plsc.load_gather(key_vm, [jv])
                    l = xv - x0s
                    t = yv - y0s
                    r = x1s - xv
                    bb = y1s - yv
                    omin = jnp.minimum(jnp.minimum(l, r), jnp.minimum(t, bb))
                    omax = jnp.maximum(jnp.maximum(l, r), jnp.maximum(t, bb))
                    dmax = jnp.maximum(jnp.abs(l - r), jnp.abs(t - bb))
                    pos = ((omin > 0.0) & (omax > lov) & (omax <= hiv)
                           & (dmax < radv))
                    if valid is not None:
                        pos = pos & valid
                    return jnp.where(pos, keys, IMAX)

                base_j = gtb + first
                lastv = jnp.full((16,), gtb + last, jnp.int32)

                def jbody(i, best):
                    jv0 = jnp.full((16,), base_j + i * 2, jnp.int32)
                    jv1 = jv0 + 1
                    am0 = one(jv0, None)
                    am1 = one(jv1, jv1 < lastv)
                    return jnp.minimum(best, jnp.minimum(am0, am1))

                best = lax.fori_loop(0, (last - first + 1) // 2, jbody,
                                     jnp.full((16,), 0x7FFFFFFF, jnp.int32))
                anyv = best < IMAX
                bj = (best & 63) + gtb

                bx0 = plsc.load_gather(x0_vm, [bj])
                by0 = plsc.load_gather(y0_vm, [bj])
                bx1 = plsc.load_gather(x1_vm, [bj])
                by1 = plsc.load_gather(y1_vm, [bj])
                labg = plsc.load_gather(lab_vm, [bj])

                lg = xv - bx0
                tg = yv - by0
                rg = bx1 - xv
                bg = by1 - yv
                lrmin = jnp.minimum(lg, rg)
                lrmax = jnp.maximum(lg, rg)
                tbmin = jnp.minimum(tg, bg)
                tbmax = jnp.maximum(tg, bg)
                ratio = (lrmin * tbmin) / (lrmax * tbmax + 1e-10)
                rs = jnp.where(anyv, ratio, 1.0)
                rs = jnp.maximum(rs, 1e-30)
                ii = lax.bitcast_convert_type(rs, jnp.int32)
                r0 = lax.bitcast_convert_type(0x5F3759DF - (ii >> 1),
                                              jnp.float32)
                for _ in range(3):
                    r0 = r0 * (1.5 - 0.5 * rs * r0 * r0)
                sq = rs * r0

                cls_o[osl] = jnp.where(anyv, labg, -1)
                cnt_o[osl] = jnp.where(anyv, sq, -1.0)
                rl_o[osl] = jnp.where(anyv, lg, -1.0)
                rt_o[osl] = jnp.where(anyv, tg, -1.0)
                rr_o[osl] = jnp.where(anyv, rg, -1.0)
                rb_o[osl] = jnp.where(anyv, bg, -1.0)

            return 0

        lax.fori_loop(0, KVECS, kbody, 0)
        return 0

    lax.fori_loop(0, B, bbody, 0)

    out_h = []
    for b in range(B):
        src = pl.ds(b * STRIPE, STRIPE)
        dst = pl.ds(b * PAD_A + abase, STRIPE)
        out_h.append(pltpu.async_copy(cls_o.at[src], cls_hbm.at[dst], sem))
        out_h.append(pltpu.async_copy(cnt_o.at[src], cnt_hbm.at[dst], sem))
        for c, r_o in enumerate((rl_o, rt_o, rr_o, rb_o)):
            out_h.append(pltpu.async_copy(
                r_o.at[src],
                reg_hbm.at[pl.ds(c * B * PAD_A + b * PAD_A + abase, STRIPE)],
                sem))
    for h in out_h:
        h.wait()


@jax.jit
def _assign(boxes_t, labels):
    f = pl.kernel(
        _body,
        out_type=[
            jax.ShapeDtypeStruct((B * PAD_A,), jnp.int32),
            jax.ShapeDtypeStruct((B * PAD_A,), jnp.float32),
            jax.ShapeDtypeStruct((4 * B * PAD_A,), jnp.float32),
        ],
        mesh=plsc.VectorSubcoreMesh(core_axis_name="c", subcore_axis_name="s",
                                    num_cores=2, num_subcores=16),
        compiler_params=pltpu.CompilerParams(needs_layout_passes=False),
        scratch_types=(
            [pltpu.VMEM((5 * STRIPE,), jnp.float32),
             pltpu.VMEM((B * 4 * NGT,), jnp.float32),
             pltpu.VMEM((B * NGT,), jnp.int32)]
            + [pltpu.VMEM((B * NGT + 16,), jnp.float32)] * 4
            + [pltpu.VMEM((B * NGT + 16,), jnp.int32)]
            + [pltpu.VMEM((B * STRIPE,), jnp.int32),
               pltpu.VMEM((B * STRIPE,), jnp.float32)]
            + [pltpu.VMEM((B * STRIPE,), jnp.float32)] * 4
            + [pltpu.SemaphoreType.DMA]
        ),
    )
    return f(boxes_t.reshape(-1), labels.reshape(-1), _PARAMS)


def kernel(gt_boxes, gt_labels):
    boxes_t = jnp.transpose(gt_boxes, (0, 2, 1))
    labels = gt_labels.astype(jnp.int32)
    cls_f, cnt_f, reg_f = _assign(boxes_t, labels)
    cls_f = cls_f.reshape(B, PAD_A)
    cnt_f = cnt_f.reshape(B, PAD_A)
    reg_f = reg_f.reshape(4, B, PAD_A)

    offs = np.cumsum([0] + list(LVL_PAD))[:-1]
    cls = jnp.concatenate(
        [cls_f[:, o:o + a] for o, a in zip(offs, LVL_SIZES)], axis=1)
    cnt = jnp.concatenate(
        [cnt_f[:, o:o + a] for o, a in zip(offs, LVL_SIZES)], axis=1)
    reg = jnp.concatenate(
        [reg_f[:, :, o:o + a] for o, a in zip(offs, LVL_SIZES)], axis=2)
    cls = cls.reshape(-1, 1)
    cnt = cnt.reshape(-1, 1)
    reg = reg.reshape(4, -1).T
    return cls, cnt, reg


# speedup vs baseline: 1.0074x; 1.0013x over previous
import numpy as np
import jax
import jax.numpy as jnp
from jax import lax
from jax.experimental import pallas as pl
from jax.experimental.pallas import tpu as pltpu
from jax.experimental.pallas import tpu_sc as plsc

IMG_SIZE = 640
STRIDES = (8, 16, 32, 64, 128)
LIMITS = ((-1.0, 64.0), (64.0, 128.0), (128.0, 256.0), (256.0, 512.0),
          (512.0, 999999.0))
RATIO = 1.5

B = 16
NGT = 64
LVL_SIZES = tuple((IMG_SIZE // s) * (IMG_SIZE // s) for s in STRIDES)
LVL_PAD = tuple(-(-a // 16) * 16 for a in LVL_SIZES)
NWORK = 32
TOT_VECS = -(-sum(LVL_PAD) // (16 * NWORK)) * NWORK
PAD_A = TOT_VECS * 16
KVECS = TOT_VECS // NWORK
STRIPE = KVECS * 16
VEC_BOUNDS = tuple(np.cumsum([p // 16 for p in LVL_PAD]))
IMAX = 0x7FFFFFFF


def _build_params():
    p = np.zeros((5, PAD_A), np.float32)
    off = 0
    for s, (lo, hi), a, ap in zip(STRIDES, LIMITS, LVL_SIZES, LVL_PAD):
        w = IMG_SIZE // s
        idx = np.arange(a)
        p[0, off:off + a] = (idx % w) * s + s // 2
        p[1, off:off + a] = (idx // w) * s + s // 2
        p[2, off:off + a] = lo
        p[3, off:off + a] = hi
        p[4, off:off + a] = 2.0 * s * RATIO
        off += ap
    return p.reshape(-1)


_PARAMS = _build_params()


def _body(boxes_hbm, labels_hbm, params_hbm,
          cls_hbm, cnt_hbm, reg_hbm,
          params_vm, box_vm, lab_vm,
          x0_vm, y0_vm, x1_vm, y1_vm, key_vm,
          cls_o, cnt_o, rl_o, rt_o, rr_o, rb_o, sem):
    w = lax.axis_index("s") * 2 + lax.axis_index("c")
    abase = w * STRIPE

    in_h = [pltpu.async_copy(params_hbm.at[pl.ds(i * PAD_A + abase, STRIPE)],
                             params_vm.at[pl.ds(i * STRIPE, STRIPE)], sem)
            for i in range(5)]
    in_h.append(pltpu.async_copy(boxes_hbm, box_vm, sem))
    in_h.append(pltpu.async_copy(labels_hbm, lab_vm, sem))
    for h in in_h:
        h.wait()

    mone_i = jnp.full((16,), -1, jnp.int32)
    mone_f = jnp.full((16,), -1.0, jnp.float32)

    def fbody(v, _):
        fsl = pl.ds(v * 16, 16)
        cls_o[fsl] = mone_i
        cnt_o[fsl] = mone_f
        rl_o[fsl] = mone_f
        rt_o[fsl] = mone_f
        rr_o[fsl] = mone_f
        rb_o[fsl] = mone_f
        return 0

    lax.fori_loop(0, B * KVECS, fbody, 0)

    los = [jnp.float32(lo) for lo, _ in LIMITS] + [jnp.float32(0.0)]
    his2 = [jnp.float32(2.0 * hi) for _, hi in LIMITS] + [jnp.float32(0.0)]

    def bbody(b, _):
        gtb = b * NGT
        ms, rms = [], []
        for i in range(NGT // 16):
            sl = pl.ds(gtb + i * 16, 16)
            x0 = box_vm[pl.ds(b * 4 * NGT + 0 * NGT + i * 16, 16)]
            y0 = box_vm[pl.ds(b * 4 * NGT + 1 * NGT + i * 16, 16)]
            wv = box_vm[pl.ds(b * 4 * NGT + 2 * NGT + i * 16, 16)]
            hv = box_vm[pl.ds(b * 4 * NGT + 3 * NGT + i * 16, 16)]
            x0_vm[sl] = x0
            y0_vm[sl] = y0
            x1_vm[sl] = x0 + wv
            y1_vm[sl] = y0 + hv
            jidx = lax.iota(jnp.int32, 16) + (i * 16)
            key_vm[sl] = (wv * hv).astype(jnp.int32) * 64 + jidx
            m = jnp.maximum(wv, hv)
            ms.append(m)
            rms.append(lax.rev(m, (0,)))

        firsts, lasts = [], []
        for lvl in range(5):
            lo, hi2 = los[lvl], his2[lvl]
            fv, lv = [], []
            for c in range(NGT // 16):
                cand = (ms[c] > lo) & (ms[c] <= hi2)
                fc = plsc.all_reduce_ffs(cand)
                fv.append(jnp.where(fc >= 16, NGT, fc + (c * 16)))
                candr = (rms[c] > lo) & (rms[c] <= hi2)
                fr = plsc.all_reduce_ffs(candr)
                lv.append(jnp.where(fr >= 16, 0, (c * 16 + 16) - fr))
            firstv = jnp.minimum(jnp.minimum(fv[0], fv[1]),
                                 jnp.minimum(fv[2], fv[3]))
            lastv = jnp.maximum(jnp.maximum(lv[0], lv[1]),
                                jnp.maximum(lv[2], lv[3]))
            firsts.append(firstv[0])
            lasts.append(lastv[0])
        firsts.append(jnp.int32(0))
        lasts.append(jnp.int32(0))

        def kbody(k, _):
            g = w * KVECS + k
            lvl = ((g >= VEC_BOUNDS[0]).astype(jnp.int32)
                   + (g >= VEC_BOUNDS[1]) + (g >= VEC_BOUNDS[2])
                   + (g >= VEC_BOUNDS[3]) + (g >= VEC_BOUNDS[4]))
            first = firsts[5]
            last = lasts[5]
            for i in range(4, -1, -1):
                first = jnp.where(lvl == i, firsts[i], first)
                last = jnp.where(lvl == i, lasts[i], last)

            osl = pl.ds(b * STRIPE + k * 16, 16)

            @pl.when(last > first)
            def _work():
                xv = params_vm[pl.ds(0 * STRIPE + k * 16, 16)]
                yv = params_vm[pl.ds(1 * STRIPE + k * 16, 16)]
                lov = params_vm[pl.ds(2 * STRIPE + k * 16, 16)]
                hiv = params_vm[pl.ds(3 * STRIPE + k * 16, 16)]
                radv = params_vm[pl.ds(4 * STRIPE + k * 16, 16)]

                def one(jv, valid):
                    x0s = plsc.load_gather(x0_vm, [jv])
                    y0s = plsc.load_gather(y0_vm, [jv])
                    x1s = plsc.load_gather(x1_vm, [jv])
                    y1s = plsc.load_gather(y1_vm, [jv])
                    keys = plsc.load_gather(key_vm, [jv])
                    l = xv - x0s
                    t = yv - y0s
                    r = x1s - xv
                    bb = y1s - yv
                    omin = jnp.minimum(jnp.minimum(l, r), jnp.minimum(t, bb))
                    omax = jnp.maximum(jnp.maximum(l, r), jnp.maximum(t, bb))
                    dmax = jnp.maximum(jnp.abs(l - r), jnp.abs(t - bb))
                    pos = ((omin > 0.0) & (omax > lov) & (omax <= hiv)
                           & (dmax < radv))
                    if valid is not None:
                        pos = pos & valid
                    return jnp.where(pos, keys, IMAX)

                base_j = gtb + first
                lastv = jnp.full((16,), gtb + last, jnp.int32)

                def jbody(i, best):
                    jv0 = jnp.full((16,), base_j + i * 2, jnp.int32)
                    jv1 = jv0 + 1
                    am0 = one(jv0, None)
                    am1 = one(jv1, jv1 < lastv)
                    return jnp.minimum(best, jnp.minimum(am0, am1))

                best = lax.fori_loop(0, (last - first + 1) // 2, jbody,
                                     jnp.full((16,), 0x7FFFFFFF, jnp.int32))
                anyv = best < IMAX
                bj = (best & 63) + gtb

                bx0 = plsc.load_gather(x0_vm, [bj])
                by0 = plsc.load_gather(y0_vm, [bj])
                bx1 = plsc.load_gather(x1_vm, [bj])
                by1 = plsc.load_gather(y1_vm, [bj])
                labg = plsc.load_gather(lab_vm, [bj])

                lg = xv - bx0
                tg = yv - by0
                rg = bx1 - xv
                bg = by1 - yv
                lrmin = jnp.minimum(lg, rg)
                lrmax = jnp.maximum(lg, rg)
                tbmin = jnp.minimum(tg, bg)
                tbmax = jnp.maximum(tg, bg)
                ratio = (lrmin * tbmin) / (lrmax * tbmax + 1e-10)
                rs = jnp.where(anyv, ratio, 1.0)
                rs = jnp.maximum(rs, 1e-30)
                ii = lax.bitcast_convert_type(rs, jnp.int32)
                r0 = lax.bitcast_convert_type(0x5F3759DF - (ii >> 1),
                                              jnp.float32)
                for _ in range(3):
                    r0 = r0 * (1.5 - 0.5 * rs * r0 * r0)
                sq = rs * r0

                cls_o[osl] = jnp.where(anyv, labg, -1)
                cnt_o[osl] = jnp.where(anyv, sq, -1.0)
                rl_o[osl] = jnp.where(anyv, lg, -1.0)
                rt_o[osl] = jnp.where(anyv, tg, -1.0)
                rr_o[osl] = jnp.where(anyv, rg, -1.0)
                rb_o[osl] = jnp.where(anyv, bg, -1.0)

            return 0

        lax.fori_loop(0, KVECS, kbody, 0)
        return 0

    lax.fori_loop(0, B, bbody, 0)

    out_h = []
    for b in range(B):
        src = pl.ds(b * STRIPE, STRIPE)
        dst = pl.ds(b * PAD_A + abase, STRIPE)
        out_h.append(pltpu.async_copy(cls_o.at[src], cls_hbm.at[dst], sem))
        out_h.append(pltpu.async_copy(cnt_o.at[src], cnt_hbm.at[dst], sem))
        for c, r_o in enumerate((rl_o, rt_o, rr_o, rb_o)):
            out_h.append(pltpu.async_copy(
                r_o.at[src],
                reg_hbm.at[pl.ds(c * B * PAD_A + b * PAD_A + abase, STRIPE)],
                sem))
    for h in out_h:
        h.wait()


@jax.jit
def _assign(boxes_t, labels):
    f = pl.kernel(
        _body,
        out_type=[
            jax.ShapeDtypeStruct((B * PAD_A,), jnp.int32),
            jax.ShapeDtypeStruct((B * PAD_A,), jnp.float32),
            jax.ShapeDtypeStruct((4 * B * PAD_A,), jnp.float32),
        ],
        mesh=plsc.VectorSubcoreMesh(core_axis_name="c", subcore_axis_name="s",
                                    num_cores=2, num_subcores=16),
        compiler_params=pltpu.CompilerParams(needs_layout_passes=False),
        scratch_types=(
            [pltpu.VMEM((5 * STRIPE,), jnp.float32),
             pltpu.VMEM((B * 4 * NGT,), jnp.float32),
             pltpu.VMEM((B * NGT,), jnp.int32)]
            + [pltpu.VMEM((B * NGT + 16,), jnp.float32)] * 4
            + [pltpu.VMEM((B * NGT + 16,), jnp.int32)]
            + [pltpu.VMEM((B * STRIPE,), jnp.int32),
               pltpu.VMEM((B * STRIPE,), jnp.float32)]
            + [pltpu.VMEM((B * STRIPE,), jnp.float32)] * 4
            + [pltpu.SemaphoreType.DMA]
        ),
    )
    return f(boxes_t.reshape(-1), labels.reshape(-1), _PARAMS)


def kernel(gt_boxes, gt_labels):
    boxes_t = jnp.transpose(gt_boxes, (0, 2, 1))
    labels = gt_labels.astype(jnp.int32)
    cls_f, cnt_f, reg_f = _assign(boxes_t, labels)
    cls_f = cls_f.reshape(B, PAD_A)
    cnt_f = cnt_f.reshape(B, PAD_A)
    reg_f = reg_f.reshape(4, B, PAD_A)

    offs = np.cumsum([0] + list(LVL_PAD))[:-1]
    cls = jnp.concatenate(
        [cls_f[:, o:o + a] for o, a in zip(offs, LVL_SIZES)], axis=1)
    cnt = jnp.concatenate(
        [cnt_f[:, o:o + a] for o, a in zip(offs, LVL_SIZES)], axis=1)
    reg = jnp.concatenate(
        [reg_f[:, :, o:o + a] for o, a in zip(offs, LVL_SIZES)], axis=2)
    cls = cls.reshape(-1, 1)
    cnt = cnt.reshape(-1, 1)
    reg = reg.reshape(4, -1).T
    return cls, cnt, reg
